# baseline (speedup 1.0000x reference)
import functools
from math import gcd as _gcd

import jax
import jax.numpy as jnp
from jax import lax
from jax.experimental import pallas as pl
from jax.experimental.pallas import tpu as pltpu
from jax.experimental.pallas import tpu_sc as plsc

NC = 2
NS = 16
NW = NC * NS
L = 72
_D = 5
_DI = 10
_AI = 4
_AG = 2
_S = 3
_SD = 6
_DD = 12
_AD = 6
LD = 128


def _deg_body(ei_hbm, zeros_hbm, out_hbm, dst_v, ones_v, deg_sh, isem, dsem,
              NCH, SLAB):
    c = lax.axis_index("c")
    s = lax.axis_index("s")
    w = c * NS + s
    nbase = NCH // NW
    extra = NCH - nbase * NW
    hi = nbase + jnp.where(w < extra, 1, 0)
    pltpu.sync_copy(zeros_hbm, deg_sh.at[pl.ds(s * SLAB, SLAB)])
    for k in range(LD // 16):
        ones_v[pl.ds(k * 16, 16)] = jnp.ones((16,), jnp.float32)
    plsc.subcore_barrier()

    def chunk_src(t):
        return ei_hbm.at[pl.ds(0, 2), pl.ds((w + NW * t) * LD, LD)]

    for j in range(_AD):
        @pl.when(j <= hi - 1)
        def _(j=j):
            pltpu.async_copy(chunk_src(j), dst_v.at[j % _DD], isem[j % _DD])

    U = _SD * _DD // _gcd(_SD, _DD)

    def body(j, carry):
        for u in range(U):

            @pl.when(j % U == u)
            def _(u=u):
                p = u % _SD
                ui = u % _DD

                @pl.when(j >= _SD)
                def _():
                    pltpu.make_async_copy(
                        ones_v, deg_sh.at[dst_v.at[(u - _SD) % _DD, 1]],
                        dsem[p]).wait()

                @pl.when(j + _AD <= hi - 1)
                def _():
                    si = (u + _AD) % _DD
                    pltpu.async_copy(chunk_src(j + _AD), dst_v.at[si],
                                     isem[si])

                pltpu.make_async_copy(chunk_src(j), dst_v.at[ui],
                                      isem[ui]).wait()
                pltpu.async_copy(ones_v, deg_sh.at[dst_v.at[ui, 1]], dsem[p],
                                 add=True)

        return carry

    lax.fori_loop(0, hi, body, 0)
    for p in range(_SD):
        @pl.when(p <= hi - 1)
        def _(p=p):
            pltpu.make_async_copy(ones_v, deg_sh.at[dst_v.at[0, 1]],
                                  dsem[p]).wait()
    plsc.subcore_barrier()
    pltpu.sync_copy(deg_sh.at[pl.ds(s * SLAB, SLAB)],
                    out_hbm.at[c, pl.ds(s * SLAB, SLAB)])


def _edge_body(hp_hbm, ei_hbm, zeros_hbm, out_hbm,
               idx_v, rows, acc_sh, isem, gsem, ssem, K, SLAB, EP):
    c = lax.axis_index("c")
    s = lax.axis_index("s")
    w = c * NS + s
    base = w * (K * L)
    pltpu.sync_copy(zeros_hbm, acc_sh.at[pl.ds(s * SLAB, SLAB)])
    plsc.subcore_barrier()

    def load_idx(j, si):
        pltpu.async_copy(ei_hbm.at[pl.ds(base + j * L, L)],
                         idx_v.at[si, 0], isem[si])
        pltpu.async_copy(ei_hbm.at[pl.ds(EP + base + j * L, L)],
                         idx_v.at[si, 1], isem[si])

    def wait_idx(j, si):
        pltpu.make_async_copy(ei_hbm.at[pl.ds(base + j * L, L)],
                              idx_v.at[si, 0], isem[si]).wait()
        pltpu.make_async_copy(ei_hbm.at[pl.ds(EP + base + j * L, L)],
                              idx_v.at[si, 1], isem[si]).wait()

    for j in range(min(_AI, K)):
        load_idx(j, j % _DI)
    for j in range(min(_AG, K)):
        si = j % _DI
        wait_idx(j, si)
        pltpu.async_copy(hp_hbm.at[idx_v.at[si, 0]], rows[j % _D],
                         gsem[j % _D])

    U = _D * _DI // _gcd(_D, _DI)

    def body(j, carry):
        for u in range(U):

            @pl.when(j % U == u)
            def _(u=u):
                p = u % _D

                @pl.when(j >= _S)
                def _():
                    q = (u - _S) % _D
                    qi = (u - _S) % _DI
                    pltpu.make_async_copy(
                        rows[q], acc_sh.at[idx_v.at[qi, 1]], ssem[q]).wait()

                @pl.when(j + _AI <= K - 1)
                def _():
                    load_idx(j + _AI, (u + _AI) % _DI)

                @pl.when(j + _AG <= K - 1)
                def _():
                    sg = (u + _AG) % _DI
                    rq = (u + _AG) % _D
                    wait_idx(j + _AG, sg)
                    pltpu.async_copy(hp_hbm.at[idx_v.at[sg, 0]], rows[rq],
                                     gsem[rq])

                ui = u % _DI
                pltpu.make_async_copy(hp_hbm.at[idx_v.at[ui, 0]], rows[p],
                                      gsem[p]).wait()
                pltpu.async_copy(rows[p], acc_sh.at[idx_v.at[ui, 1]], ssem[p],
                                 add=True)

        return carry

    lax.fori_loop(0, K, body, 0)
    for j in range(max(K - _S, 0), K):
        pltpu.make_async_copy(rows[j % _D], acc_sh.at[idx_v.at[j % _DI, 1]],
                              ssem[j % _D]).wait()
    plsc.subcore_barrier()
    pltpu.sync_copy(acc_sh.at[pl.ds(s * SLAB, SLAB)],
                    out_hbm.at[c, pl.ds(s * SLAB, SLAB)])


def _dis_col(degs_ref):
    d2 = lax.dot_general(degs_ref[...], jnp.eye(2, dtype=jnp.float32),
                         (((0,), (0,)), ((), ())),
                         preferred_element_type=jnp.float32)
    return lax.rsqrt(d2[:, 0:1] + d2[:, 1:2] + 1.0)


def _mm_body(x_ref, wg_ref, h_ref):
    h_ref[...] = jnp.dot(x_ref[...], wg_ref[...],
                         preferred_element_type=jnp.float32)


def _scale_body(h_ref, degs_ref, hp_ref):
    hp_ref[...] = h_ref[...] * _dis_col(degs_ref)


def _tail_body(acc_ref, hp_ref, degs_ref, bg_ref, w1_ref, b1_ref, w2_ref,
               b2_ref, y_ref):
    agg = acc_ref[0] + acc_ref[1] + hp_ref[...]
    m = jnp.maximum(agg * _dis_col(degs_ref) + bg_ref[...], 0.0)
    h2 = jnp.maximum(
        jnp.dot(m, w1_ref[...], preferred_element_type=jnp.float32)
        + b1_ref[...], 0.0)
    y_ref[...] = (jnp.dot(h2, w2_ref[...], preferred_element_type=jnp.float32)
                  + b2_ref[...])


def kernel(x, edge_index, Wg, bg, W1, b1, W2, b2):
    N, F = x.shape
    E = edge_index.shape[1]
    K = -(-E // (NW * L))
    EP = NW * K * L
    NP = ((N + NS * 8 - 1) // (NS * 8)) * (NS * 8) + NS * 8
    SLAB = NP // NS

    ei = edge_index.astype(jnp.int32)
    if EP != E:
        padcol = jnp.stack([jnp.zeros((EP - E,), jnp.int32),
                            jnp.full((EP - E,), N, jnp.int32)])
        ei = jnp.concatenate([ei, padcol], axis=1)
    ei1 = ei.reshape(-1)
    zeros1 = jnp.zeros((SLAB,), jnp.float32)
    zeros2 = jnp.zeros((SLAB, F), jnp.float32)

    mesh = plsc.VectorSubcoreMesh(core_axis_name="c", subcore_axis_name="s")

    if E % LD:
        padd = LD - E % LD
        ei_deg = jnp.concatenate(
            [ei, jnp.stack([jnp.zeros((padd,), jnp.int32),
                            jnp.full((padd,), N, jnp.int32)])], axis=1)
    else:
        ei_deg = ei
    NCH = ei_deg.shape[1] // LD
    deg_call = pl.kernel(
        functools.partial(_deg_body, NCH=NCH, SLAB=SLAB),
        out_type=jax.ShapeDtypeStruct((NC, NP), jnp.float32),
        mesh=mesh,
        scratch_types=[
            pltpu.VMEM((_DD, 2, LD), jnp.int32),
            pltpu.VMEM((LD,), jnp.float32),
            pltpu.VMEM_SHARED((NP,), jnp.float32),
            tuple(pltpu.SemaphoreType.DMA for _ in range(_DD)),
            tuple(pltpu.SemaphoreType.DMA for _ in range(_SD)),
        ],
    )
    degs = deg_call(ei_deg, zeros1)

    RB = 2048
    grid = -(-N // RB)
    h = pl.pallas_call(
        _mm_body,
        grid=(grid,),
        in_specs=[
            pl.BlockSpec((RB, F), lambda i: (i, 0)),
            pl.BlockSpec((F, F), lambda i: (0, 0)),
        ],
        out_specs=pl.BlockSpec((RB, F), lambda i: (i, 0)),
        out_shape=jax.ShapeDtypeStruct((N, F), jnp.float32),
    )(x, Wg)
    hp = pl.pallas_call(
        _scale_body,
        grid=(grid,),
        in_specs=[
            pl.BlockSpec((RB, F), lambda i: (i, 0)),
            pl.BlockSpec((NC, RB), lambda i: (0, i)),
        ],
        out_specs=pl.BlockSpec((RB, F), lambda i: (i, 0)),
        out_shape=jax.ShapeDtypeStruct((N, F), jnp.float32),
    )(h, degs)

    edge_call = pl.kernel(
        functools.partial(_edge_body, K=K, SLAB=SLAB, EP=EP),
        out_type=jax.ShapeDtypeStruct((NC, NP, F), jnp.float32),
        mesh=mesh,
        scratch_types=[
            pltpu.VMEM((_DI, 2, L), jnp.int32),
            tuple(pltpu.VMEM((L, F), jnp.float32) for _ in range(_D)),
            pltpu.VMEM_SHARED((NP, F), jnp.float32),
            tuple(pltpu.SemaphoreType.DMA for _ in range(_DI)),
            tuple(pltpu.SemaphoreType.DMA for _ in range(_D)),
            tuple(pltpu.SemaphoreType.DMA for _ in range(_D)),
        ],
    )
    acc = edge_call(hp, ei1, zeros2)

    y = pl.pallas_call(
        _tail_body,
        grid=(grid,),
        in_specs=[
            pl.BlockSpec((NC, RB, F), lambda i: (0, i, 0)),
            pl.BlockSpec((RB, F), lambda i: (i, 0)),
            pl.BlockSpec((NC, RB), lambda i: (0, i)),
            pl.BlockSpec((1, F), lambda i: (0, 0)),
            pl.BlockSpec((F, F), lambda i: (0, 0)),
            pl.BlockSpec((1, F), lambda i: (0, 0)),
            pl.BlockSpec((F, 1), lambda i: (0, 0)),
            pl.BlockSpec((1, 1), lambda i: (0, 0)),
        ],
        out_specs=pl.BlockSpec((RB, 1), lambda i: (i, 0)),
        out_shape=jax.ShapeDtypeStruct((N, 1), jnp.float32),
    )(acc, hp, degs, bg.reshape(1, F), W1, b1.reshape(1, F),
      W2, b2.reshape(1, 1))
    return y


# speedup vs baseline: 1.0802x; 1.0802x over previous
import functools
from math import gcd as _gcd

import jax
import jax.numpy as jnp
from jax import lax
from jax.experimental import pallas as pl
from jax.experimental.pallas import tpu as pltpu
from jax.experimental.pallas import tpu_sc as plsc

NC = 2
NS = 16
NW = NC * NS
L = 80
_D = 4
_DI = 8
_AI = 4
_AG = 2
_S = 2
_SD = 6
_DD = 12
_AD = 6
LD = 128


def _deg_body(ei_hbm, zeros_hbm, out_hbm, dst_v, ones_v, deg_sh, isem, dsem,
              NCH, SLAB):
    c = lax.axis_index("c")
    s = lax.axis_index("s")
    w = c * NS + s
    nbase = NCH // NW
    extra = NCH - nbase * NW
    hi = nbase + jnp.where(w < extra, 1, 0)
    pltpu.sync_copy(zeros_hbm, deg_sh.at[pl.ds(s * SLAB, SLAB)])
    for k in range(LD // 16):
        ones_v[pl.ds(k * 16, 16)] = jnp.ones((16,), jnp.float32)
    plsc.subcore_barrier()

    def chunk_src(t):
        return ei_hbm.at[pl.ds(0, 2), pl.ds((w + NW * t) * LD, LD)]

    for j in range(_AD):
        @pl.when(j <= hi - 1)
        def _(j=j):
            pltpu.async_copy(chunk_src(j), dst_v.at[j % _DD], isem[j % _DD])

    U = _SD * _DD // _gcd(_SD, _DD)

    def body(j, carry):
        for u in range(U):

            @pl.when(j % U == u)
            def _(u=u):
                p = u % _SD
                ui = u % _DD

                @pl.when(j >= _SD)
                def _():
                    pltpu.make_async_copy(
                        ones_v, deg_sh.at[dst_v.at[(u - _SD) % _DD, 1]],
                        dsem[p]).wait()

                @pl.when(j + _AD <= hi - 1)
                def _():
                    si = (u + _AD) % _DD
                    pltpu.async_copy(chunk_src(j + _AD), dst_v.at[si],
                                     isem[si])

                pltpu.make_async_copy(chunk_src(j), dst_v.at[ui],
                                      isem[ui]).wait()
                pltpu.async_copy(ones_v, deg_sh.at[dst_v.at[ui, 1]], dsem[p],
                                 add=True)

        return carry

    lax.fori_loop(0, hi, body, 0)
    for p in range(_SD):
        @pl.when(p <= hi - 1)
        def _(p=p):
            pltpu.make_async_copy(ones_v, deg_sh.at[dst_v.at[0, 1]],
                                  dsem[p]).wait()
    plsc.subcore_barrier()
    pltpu.sync_copy(deg_sh.at[pl.ds(s * SLAB, SLAB)],
                    out_hbm.at[c, pl.ds(s * SLAB, SLAB)])


def _edge_body(hp_hbm, ei_hbm, zeros_hbm, out_hbm,
               idx_v, rows, acc_sh, isem, gsem, ssem, K, SLAB, EP):
    c = lax.axis_index("c")
    s = lax.axis_index("s")
    w = c * NS + s
    base = w * (K * L)
    pltpu.sync_copy(zeros_hbm, acc_sh.at[pl.ds(s * SLAB, SLAB)])
    plsc.subcore_barrier()

    def load_idx(j, si):
        pltpu.async_copy(ei_hbm.at[pl.ds(base + j * L, L)],
                         idx_v.at[si, 0], isem[si])
        pltpu.async_copy(ei_hbm.at[pl.ds(EP + base + j * L, L)],
                         idx_v.at[si, 1], isem[si])

    def wait_idx(j, si):
        pltpu.make_async_copy(ei_hbm.at[pl.ds(base + j * L, L)],
                              idx_v.at[si, 0], isem[si]).wait()
        pltpu.make_async_copy(ei_hbm.at[pl.ds(EP + base + j * L, L)],
                              idx_v.at[si, 1], isem[si]).wait()

    for j in range(min(_AI, K)):
        load_idx(j, j % _DI)
    for j in range(min(_AG, K)):
        si = j % _DI
        wait_idx(j, si)
        pltpu.async_copy(hp_hbm.at[idx_v.at[si, 0]], rows[j % _D],
                         gsem[j % _D])

    U = _D * _DI // _gcd(_D, _DI)

    def body(j, carry):
        for u in range(U):

            @pl.when(j % U == u)
            def _(u=u):
                p = u % _D

                @pl.when(j >= _S)
                def _():
                    q = (u - _S) % _D
                    qi = (u - _S) % _DI
                    pltpu.make_async_copy(
                        rows[q], acc_sh.at[idx_v.at[qi, 1]], ssem[q]).wait()

                @pl.when(j + _AI <= K - 1)
                def _():
                    load_idx(j + _AI, (u + _AI) % _DI)

                @pl.when(j + _AG <= K - 1)
                def _():
                    sg = (u + _AG) % _DI
                    rq = (u + _AG) % _D
                    wait_idx(j + _AG, sg)
                    pltpu.async_copy(hp_hbm.at[idx_v.at[sg, 0]], rows[rq],
                                     gsem[rq])

                ui = u % _DI
                pltpu.make_async_copy(hp_hbm.at[idx_v.at[ui, 0]], rows[p],
                                      gsem[p]).wait()
                pltpu.async_copy(rows[p], acc_sh.at[idx_v.at[ui, 1]], ssem[p],
                                 add=True)

        return carry

    lax.fori_loop(0, K, body, 0)
    for j in range(max(K - _S, 0), K):
        pltpu.make_async_copy(rows[j % _D], acc_sh.at[idx_v.at[j % _DI, 1]],
                              ssem[j % _D]).wait()
    plsc.subcore_barrier()
    pltpu.sync_copy(acc_sh.at[pl.ds(s * SLAB, SLAB)],
                    out_hbm.at[c, pl.ds(s * SLAB, SLAB)])


def _dis_col(degs_ref):
    d2 = lax.dot_general(degs_ref[...], jnp.eye(2, dtype=jnp.float32),
                         (((0,), (0,)), ((), ())),
                         preferred_element_type=jnp.float32)
    return lax.rsqrt(d2[:, 0:1] + d2[:, 1:2] + 1.0)


def _mm_body(x_ref, wg_ref, h_ref):
    h_ref[...] = jnp.dot(x_ref[...], wg_ref[...],
                         preferred_element_type=jnp.float32)


def _scale_body(h_ref, degs_ref, hp_ref):
    hp_ref[...] = h_ref[...] * _dis_col(degs_ref)


def _tail_body(acc_ref, hp_ref, degs_ref, bg_ref, w1_ref, b1_ref, w2_ref,
               b2_ref, y_ref):
    agg = acc_ref[0] + acc_ref[1] + hp_ref[...]
    m = jnp.maximum(agg * _dis_col(degs_ref) + bg_ref[...], 0.0)
    h2 = jnp.maximum(
        jnp.dot(m, w1_ref[...], preferred_element_type=jnp.float32)
        + b1_ref[...], 0.0)
    y_ref[...] = (jnp.dot(h2, w2_ref[...], preferred_element_type=jnp.float32)
                  + b2_ref[...])


def kernel(x, edge_index, Wg, bg, W1, b1, W2, b2):
    N, F = x.shape
    E = edge_index.shape[1]
    K = -(-E // (NW * L))
    EP = NW * K * L
    NP = ((N + NS * 8 - 1) // (NS * 8)) * (NS * 8) + NS * 8
    SLAB = NP // NS

    ei = edge_index.astype(jnp.int32)
    if EP != E:
        padcol = jnp.stack([jnp.zeros((EP - E,), jnp.int32),
                            jnp.full((EP - E,), N, jnp.int32)])
        ei = jnp.concatenate([ei, padcol], axis=1)
    ei1 = ei.reshape(-1)
    zeros1 = jnp.zeros((SLAB,), jnp.float32)
    zeros2 = jnp.zeros((SLAB, F), jnp.float32)

    mesh = plsc.VectorSubcoreMesh(core_axis_name="c", subcore_axis_name="s")

    if E % LD:
        padd = LD - E % LD
        ei_deg = jnp.concatenate(
            [ei, jnp.stack([jnp.zeros((padd,), jnp.int32),
                            jnp.full((padd,), N, jnp.int32)])], axis=1)
    else:
        ei_deg = ei
    NCH = ei_deg.shape[1] // LD
    deg_call = pl.kernel(
        functools.partial(_deg_body, NCH=NCH, SLAB=SLAB),
        out_type=jax.ShapeDtypeStruct((NC, NP), jnp.float32),
        mesh=mesh,
        scratch_types=[
            pltpu.VMEM((_DD, 2, LD), jnp.int32),
            pltpu.VMEM((LD,), jnp.float32),
            pltpu.VMEM_SHARED((NP,), jnp.float32),
            tuple(pltpu.SemaphoreType.DMA for _ in range(_DD)),
            tuple(pltpu.SemaphoreType.DMA for _ in range(_SD)),
        ],
    )
    degs = deg_call(ei_deg, zeros1)

    RB = 2048
    grid = -(-N // RB)
    h = pl.pallas_call(
        _mm_body,
        grid=(grid,),
        in_specs=[
            pl.BlockSpec((RB, F), lambda i: (i, 0)),
            pl.BlockSpec((F, F), lambda i: (0, 0)),
        ],
        out_specs=pl.BlockSpec((RB, F), lambda i: (i, 0)),
        out_shape=jax.ShapeDtypeStruct((N, F), jnp.float32),
    )(x, Wg)
    hp = pl.pallas_call(
        _scale_body,
        grid=(grid,),
        in_specs=[
            pl.BlockSpec((RB, F), lambda i: (i, 0)),
            pl.BlockSpec((NC, RB), lambda i: (0, i)),
        ],
        out_specs=pl.BlockSpec((RB, F), lambda i: (i, 0)),
        out_shape=jax.ShapeDtypeStruct((N, F), jnp.float32),
    )(h, degs)

    edge_call = pl.kernel(
        functools.partial(_edge_body, K=K, SLAB=SLAB, EP=EP),
        out_type=jax.ShapeDtypeStruct((NC, NP, F), jnp.float32),
        mesh=mesh,
        scratch_types=[
            pltpu.VMEM((_DI, 2, L), jnp.int32),
            tuple(pltpu.VMEM((L, F), jnp.float32) for _ in range(_D)),
            pltpu.VMEM_SHARED((NP, F), jnp.float32),
            tuple(pltpu.SemaphoreType.DMA for _ in range(_DI)),
            tuple(pltpu.SemaphoreType.DMA for _ in range(_D)),
            tuple(pltpu.SemaphoreType.DMA for _ in range(_D)),
        ],
    )
    acc = edge_call(hp, ei1, zeros2)

    y = pl.pallas_call(
        _tail_body,
        grid=(grid,),
        in_specs=[
            pl.BlockSpec((NC, RB, F), lambda i: (0, i, 0)),
            pl.BlockSpec((RB, F), lambda i: (i, 0)),
            pl.BlockSpec((NC, RB), lambda i: (0, i)),
            pl.BlockSpec((1, F), lambda i: (0, 0)),
            pl.BlockSpec((F, F), lambda i: (0, 0)),
            pl.BlockSpec((1, F), lambda i: (0, 0)),
            pl.BlockSpec((F, 1), lambda i: (0, 0)),
            pl.BlockSpec((1, 1), lambda i: (0, 0)),
        ],
        out_specs=pl.BlockSpec((RB, 1), lambda i: (i, 0)),
        out_shape=jax.ShapeDtypeStruct((N, 1), jnp.float32),
    )(acc, hp, degs, bg.reshape(1, F), W1, b1.reshape(1, F),
      W2, b2.reshape(1, 1))
    return y
